# baseline (speedup 1.0000x reference)
import functools

import jax
import jax.numpy as jnp
from jax import lax
from jax.experimental import pallas as pl
from jax.experimental.pallas import tpu as pltpu
from jax.experimental.pallas import tpu_sc as plsc

B = 16384
D = 128
NC = 2
NS = 16
L = 16
NW = NC * NS
BPW = B // NW
C = 128
NCHUNK = BPW // C


@functools.partial(
    pl.kernel,
    out_type=jax.ShapeDtypeStruct((B,), jnp.float32),
    mesh=plsc.VectorSubcoreMesh(core_axis_name="c", subcore_axis_name="s"),
    compiler_params=pltpu.CompilerParams(needs_layout_passes=False),
    scratch_types=[
        pltpu.VMEM((NCHUNK, C), jnp.int32),
        pltpu.VMEM((NCHUNK, C), jnp.int32),
        pltpu.VMEM((C, D), jnp.float32),
        pltpu.VMEM((C, D), jnp.float32),
        pltpu.VMEM((C, D), jnp.float32),
        pltpu.VMEM((C, D), jnp.float32),
        pltpu.VMEM((BPW + L,), jnp.float32),
        pltpu.SemaphoreType.DMA,
        pltpu.SemaphoreType.DMA,
        pltpu.SemaphoreType.DMA,
    ],
)
def _skipgram_sc(center_hbm, context_hbm, cidx_hbm, xidx_hbm, out_hbm,
                 idx_c, idx_x, cb0, cb1, xb0, xb1, outb, sem0, sem1, semi):
    wid = lax.axis_index("s") * NC + lax.axis_index("c")
    base = wid * BPW

    idx_dmas = []
    for k in range(NCHUNK):
        idx_dmas.append(pltpu.async_copy(
            cidx_hbm.at[pl.ds(base + k * C, C)], idx_c.at[k], semi))
        idx_dmas.append(pltpu.async_copy(
            xidx_hbm.at[pl.ds(base + k * C, C)], idx_x.at[k], semi))
    idx_dmas[0].wait()
    idx_dmas[1].wait()

    cbufs = (cb0, cb1)
    xbufs = (xb0, xb1)
    sems = (sem0, sem1)

    def fire(k):
        p = k % 2
        dc = pltpu.async_copy(center_hbm.at[idx_c.at[k]], cbufs[p], sems[p])
        dx = pltpu.async_copy(context_hbm.at[idx_x.at[k]], xbufs[p], sems[p])
        return dc, dx

    pending = fire(0)
    for d in idx_dmas[2:]:
        d.wait()
    iota = lax.iota(jnp.int32, L)
    last_lane = iota == (L - 1)

    for k in range(NCHUNK):
        nxt = fire(k + 1) if k + 1 < NCHUNK else None
        pending[0].wait()
        pending[1].wait()
        cb = cbufs[k % 2]
        xb = xbufs[k % 2]

        @plsc.parallel_loop(0, C, step=1, unroll=16)
        def rowbody(r, cb=cb, xb=xb, k=k):
            prods = [cb[r, pl.ds(j * L, L)] * xb[r, pl.ds(j * L, L)]
                     for j in range(D // L)]
            while len(prods) > 1:
                prods = [prods[m] + prods[m + 1]
                         for m in range(0, len(prods), 2)]
            scanv = plsc.cumsum(prods[0])
            plsc.store_compressed(outb.at[pl.ds(k * C + r, L)], scanv,
                                  mask=last_lane)

        pending = nxt

    pltpu.sync_copy(outb.at[pl.ds(0, BPW)], out_hbm.at[pl.ds(base, BPW)])


def kernel(center_table, context_table, center_word_idx, context_word_idx):
    return _skipgram_sc(center_table, context_table,
                        center_word_idx, context_word_idx)


# speedup vs baseline: 1.3465x; 1.3465x over previous
import functools

import jax
import jax.numpy as jnp
from jax import lax
from jax.experimental import pallas as pl
from jax.experimental.pallas import tpu as pltpu
from jax.experimental.pallas import tpu_sc as plsc

B = 16384
D = 128
NC = 2
NS = 16
L = 16
NW = NC * NS
BPW = B // NW
C = 128
NCHUNK = BPW // C


@functools.partial(
    pl.kernel,
    out_type=jax.ShapeDtypeStruct((B,), jnp.float32),
    mesh=plsc.VectorSubcoreMesh(core_axis_name="c", subcore_axis_name="s"),
    compiler_params=pltpu.CompilerParams(needs_layout_passes=False),
    scratch_types=[
        pltpu.VMEM((NCHUNK, C), jnp.int32),
        pltpu.VMEM((NCHUNK, C), jnp.int32),
        pltpu.VMEM((C, D), jnp.float32),
        pltpu.VMEM((C, D), jnp.float32),
        pltpu.VMEM((C, D), jnp.float32),
        pltpu.VMEM((C, D), jnp.float32),
        pltpu.VMEM((BPW + L,), jnp.float32),
        pltpu.SemaphoreType.DMA,
        pltpu.SemaphoreType.DMA,
        pltpu.SemaphoreType.DMA,
    ],
)
def _skipgram_sc(center_hbm, context_hbm, cidx_hbm, xidx_hbm, out_hbm,
                 idx_c, idx_x, cb0, cb1, xb0, xb1, outb, sem0, sem1, semi):
    wid = lax.axis_index("s") * NC + lax.axis_index("c")
    base = wid * BPW

    idx_dmas = []
    for k in range(NCHUNK):
        idx_dmas.append(pltpu.async_copy(
            cidx_hbm.at[pl.ds(base + k * C, C)], idx_c.at[k], semi))
        idx_dmas.append(pltpu.async_copy(
            xidx_hbm.at[pl.ds(base + k * C, C)], idx_x.at[k], semi))
    idx_dmas[0].wait()
    idx_dmas[1].wait()

    cbufs = (cb0, cb1)
    xbufs = (xb0, xb1)
    sems = (sem0, sem1)

    def fire(k):
        p = k % 2
        dc = pltpu.async_copy(center_hbm.at[idx_c.at[k]], cbufs[p], sems[p])
        dx = pltpu.async_copy(context_hbm.at[idx_x.at[k]], xbufs[p], sems[p])
        return dc, dx

    pending = fire(0)
    for d in idx_dmas[2:]:
        d.wait()
    iota = lax.iota(jnp.int32, L)
    last_lane = iota == (L - 1)

    for k in range(NCHUNK):
        nxt = fire(k + 1) if k + 1 < NCHUNK else None
        pending[0].wait()
        pending[1].wait()
        cb = cbufs[k % 2]
        xb = xbufs[k % 2]

        @plsc.parallel_loop(0, C, step=1, unroll=4)
        def rowbody(r, cb=cb, xb=xb, k=k):
            prods = [cb[r, pl.ds(j * L, L)] * xb[r, pl.ds(j * L, L)]
                     for j in range(D // L)]
            while len(prods) > 1:
                prods = [prods[m] + prods[m + 1]
                         for m in range(0, len(prods), 2)]
            scanv = plsc.cumsum(prods[0])
            plsc.store_compressed(outb.at[pl.ds(k * C + r, L)], scanv,
                                  mask=last_lane)

        pending = nxt

    pltpu.sync_copy(outb.at[pl.ds(0, BPW)], out_hbm.at[pl.ds(base, BPW)])


def kernel(center_table, context_table, center_word_idx, context_word_idx):
    return _skipgram_sc(center_table, context_table,
                        center_word_idx, context_word_idx)


# speedup vs baseline: 1.3653x; 1.0140x over previous
import functools

import jax
import jax.numpy as jnp
from jax import lax
from jax.experimental import pallas as pl
from jax.experimental.pallas import tpu as pltpu
from jax.experimental.pallas import tpu_sc as plsc

B = 16384
D = 128
NC = 2
NS = 16
L = 16
NW = NC * NS
BPW = B // NW
C = 128
NCHUNK = BPW // C


@functools.partial(
    pl.kernel,
    out_type=jax.ShapeDtypeStruct((B,), jnp.float32),
    mesh=plsc.VectorSubcoreMesh(core_axis_name="c", subcore_axis_name="s"),
    compiler_params=pltpu.CompilerParams(needs_layout_passes=False),
    scratch_types=[
        pltpu.VMEM((NCHUNK, C), jnp.int32),
        pltpu.VMEM((NCHUNK, C), jnp.int32),
        pltpu.VMEM((C, D), jnp.float32),
        pltpu.VMEM((C, D), jnp.float32),
        pltpu.VMEM((C, D), jnp.float32),
        pltpu.VMEM((C, D), jnp.float32),
        pltpu.VMEM((BPW + L,), jnp.float32),
        pltpu.SemaphoreType.DMA,
        pltpu.SemaphoreType.DMA,
        pltpu.SemaphoreType.DMA,
    ],
)
def _skipgram_sc(center_hbm, context_hbm, cidx_hbm, xidx_hbm, out_hbm,
                 idx_c, idx_x, cb0, cb1, xb0, xb1, outb, sem0, sem1, semi):
    wid = lax.axis_index("s") * NC + lax.axis_index("c")
    base = wid * BPW

    idx_dmas = []
    for k in range(NCHUNK):
        idx_dmas.append(pltpu.async_copy(
            cidx_hbm.at[pl.ds(base + k * C, C)], idx_c.at[k], semi))
        idx_dmas.append(pltpu.async_copy(
            xidx_hbm.at[pl.ds(base + k * C, C)], idx_x.at[k], semi))
    idx_dmas[0].wait()
    idx_dmas[1].wait()

    cbufs = (cb0, cb1)
    xbufs = (xb0, xb1)
    sems = (sem0, sem1)

    def fire(k):
        p = k % 2
        dc = pltpu.async_copy(center_hbm.at[idx_c.at[k]], cbufs[p], sems[p])
        dx = pltpu.async_copy(context_hbm.at[idx_x.at[k]], xbufs[p], sems[p])
        return dc, dx

    pending = fire(0)
    for d in idx_dmas[2:]:
        d.wait()
    iota = lax.iota(jnp.int32, L)
    last_lane = iota == (L - 1)

    for k in range(NCHUNK):
        nxt = fire(k + 1) if k + 1 < NCHUNK else None
        pending[0].wait()
        pending[1].wait()
        cb = cbufs[k % 2]
        xb = xbufs[k % 2]

        @plsc.parallel_loop(0, C, step=1, unroll=2)
        def rowbody(r, cb=cb, xb=xb, k=k):
            prods = [cb[r, pl.ds(j * L, L)] * xb[r, pl.ds(j * L, L)]
                     for j in range(D // L)]
            while len(prods) > 1:
                prods = [prods[m] + prods[m + 1]
                         for m in range(0, len(prods), 2)]
            scanv = plsc.cumsum(prods[0])
            plsc.store_compressed(outb.at[pl.ds(k * C + r, L)], scanv,
                                  mask=last_lane)

        pending = nxt

    pltpu.sync_copy(outb.at[pl.ds(0, BPW)], out_hbm.at[pl.ds(base, BPW)])


def kernel(center_table, context_table, center_word_idx, context_word_idx):
    return _skipgram_sc(center_table, context_table,
                        center_word_idx, context_word_idx)


# speedup vs baseline: 1.3756x; 1.0076x over previous
import functools

import jax
import jax.numpy as jnp
from jax import lax
from jax.experimental import pallas as pl
from jax.experimental.pallas import tpu as pltpu
from jax.experimental.pallas import tpu_sc as plsc

B = 16384
D = 128
NC = 2
NS = 16
L = 16
NW = NC * NS
BPW = B // NW
C = 128
NCHUNK = BPW // C


@functools.partial(
    pl.kernel,
    out_type=jax.ShapeDtypeStruct((B,), jnp.float32),
    mesh=plsc.VectorSubcoreMesh(core_axis_name="c", subcore_axis_name="s"),
    compiler_params=pltpu.CompilerParams(needs_layout_passes=False),
    scratch_types=[
        pltpu.VMEM((NCHUNK, C), jnp.int32),
        pltpu.VMEM((NCHUNK, C), jnp.int32),
        pltpu.VMEM((C, D), jnp.float32),
        pltpu.VMEM((C, D), jnp.float32),
        pltpu.VMEM((C, D), jnp.float32),
        pltpu.VMEM((C, D), jnp.float32),
        pltpu.VMEM((BPW + L,), jnp.float32),
        pltpu.SemaphoreType.DMA,
        pltpu.SemaphoreType.DMA,
        pltpu.SemaphoreType.DMA,
    ],
)
def _skipgram_sc(center_hbm, context_hbm, cidx_hbm, xidx_hbm, out_hbm,
                 idx_c, idx_x, cb0, cb1, xb0, xb1, outb, sem0, sem1, semi):
    wid = lax.axis_index("s") * NC + lax.axis_index("c")
    base = wid * BPW

    idx_dmas = []
    for k in range(NCHUNK):
        idx_dmas.append(pltpu.async_copy(
            cidx_hbm.at[pl.ds(base + k * C, C)], idx_c.at[k], semi))
        idx_dmas.append(pltpu.async_copy(
            xidx_hbm.at[pl.ds(base + k * C, C)], idx_x.at[k], semi))
    idx_dmas[0].wait()
    idx_dmas[1].wait()

    cbufs = (cb0, cb1)
    xbufs = (xb0, xb1)
    sems = (sem0, sem1)

    def fire(k):
        p = k % 2
        dc = pltpu.async_copy(center_hbm.at[idx_c.at[k]], cbufs[p], sems[p])
        dx = pltpu.async_copy(context_hbm.at[idx_x.at[k]], xbufs[p], sems[p])
        return dc, dx

    pending = fire(0)
    for d in idx_dmas[2:]:
        d.wait()
    iota = lax.iota(jnp.int32, L)
    last_lane = iota == (L - 1)

    for k in range(NCHUNK):
        nxt = fire(k + 1) if k + 1 < NCHUNK else None
        pending[0].wait()
        pending[1].wait()
        cb = cbufs[k % 2]
        xb = xbufs[k % 2]

        @plsc.parallel_loop(0, C, step=1, unroll=1)
        def rowbody(r, cb=cb, xb=xb, k=k):
            prods = [cb[r, pl.ds(j * L, L)] * xb[r, pl.ds(j * L, L)]
                     for j in range(D // L)]
            while len(prods) > 1:
                prods = [prods[m] + prods[m + 1]
                         for m in range(0, len(prods), 2)]
            scanv = plsc.cumsum(prods[0])
            plsc.store_compressed(outb.at[pl.ds(k * C + r, L)], scanv,
                                  mask=last_lane)

        pending = nxt

    pltpu.sync_copy(outb.at[pl.ds(0, BPW)], out_hbm.at[pl.ds(base, BPW)])


def kernel(center_table, context_table, center_word_idx, context_word_idx):
    return _skipgram_sc(center_table, context_table,
                        center_word_idx, context_word_idx)
